# Initial kernel scaffold; baseline (speedup 1.0000x reference)
import jax
import jax.numpy as jnp
from jax.experimental import pallas as pl


def kernel(img, target):
    raise NotImplementedError("write your pallas kernel here")


# speedup vs baseline: 202.4667x; 202.4667x over previous
import functools

import jax
import jax.numpy as jnp
from jax import lax
from jax.experimental import pallas as pl
from jax.experimental.pallas import tpu as pltpu
from jax.experimental.pallas import tpu_sc as plsc

L = 16
NBINS = 256
NWORKERS = 32
CHUNK = 16384


def _make_equalize(n_planes: int, n_pix: int):
    assert n_planes % NWORKERS == 0
    assert n_pix % CHUNK == 0
    planes_per_w = n_planes // NWORKERS
    nchunks = n_pix // CHUNK
    vecs = CHUNK // L

    mesh = plsc.VectorSubcoreMesh(core_axis_name="c", subcore_axis_name="s")

    @functools.partial(
        pl.kernel,
        out_type=jax.ShapeDtypeStruct((n_planes * n_pix,), jnp.float32),
        mesh=mesh,
        compiler_params=pltpu.CompilerParams(needs_layout_passes=False),
        scratch_types=[
            pltpu.VMEM((CHUNK,), jnp.float32),
            pltpu.VMEM((CHUNK,), jnp.float32),
            pltpu.VMEM((CHUNK,), jnp.float32),
            pltpu.VMEM((CHUNK,), jnp.float32),
            pltpu.VMEM((L * NBINS,), jnp.int32),
            pltpu.VMEM((NBINS,), jnp.float32),
        ],
    )
    def eq_kernel(img_hbm, out_hbm, in_a, in_b, out_a, out_b, hist, lut):
        wid = lax.axis_index("s") * 2 + lax.axis_index("c")
        iota16 = lax.iota(jnp.int32, L)
        lane_base = iota16 * NBINS
        ones = jnp.ones((L,), jnp.int32)
        ibufs = [in_a, in_b]
        obufs = [out_a, out_b]

        def per_plane(j, _):
            plane = wid * planes_per_w + j
            base = plane * n_pix

            def zero_body(t, c):
                hist[pl.ds(t * L, L)] = jnp.zeros((L,), jnp.int32)
                return c
            lax.fori_loop(0, (L * NBINS) // L, zero_body, 0)

            for c in range(nchunks):
                buf = ibufs[c % 2]
                off = pl.multiple_of(base + c * CHUNK, CHUNK)
                pltpu.sync_copy(img_hbm.at[pl.ds(off, CHUNK)], buf)

                def hist_body(i, cc):
                    v = buf[pl.ds(i * L, L)]
                    idx = jnp.clip(v, 0.0, 255.0).astype(jnp.int32)
                    plsc.addupdate_scatter(hist, [idx + lane_base], ones)
                    return cc
                lax.fori_loop(0, vecs, hist_body, 0, unroll=4)

            def merge_body(t, carry):
                csum, mvec = carry
                acc = hist[pl.ds(t * L, L)]
                for ln in range(1, L):
                    acc = acc + hist[pl.ds(ln * NBINS + t * L, L)]
                inc = plsc.cumsum(acc) + csum
                hist[pl.ds(t * L, L)] = inc - acc
                mvec = jnp.maximum(mvec, jnp.where(inc < n_pix, inc, 0))
                return (csum + jnp.sum(acc), mvec)

            _, mvec = lax.fori_loop(
                0, NBINS // L, merge_body,
                (jnp.int32(0), jnp.zeros((L,), jnp.int32)))
            m = jnp.max(mvec)

            q = (m.astype(jnp.float32) * jnp.float32(1.0 / 255.0)) \
                .astype(jnp.int32)
            q = q - jnp.where(q * 255 > m, 1, 0)
            q = q + jnp.where((q + 1) * 255 <= m, 1, 0)
            step = q
            s2 = step >> 1
            ms = jnp.maximum(step, 1)
            msf = ms.astype(jnp.float32)
            seed = lax.bitcast_convert_type(
                jnp.int32(0x7EF477D5)
                - lax.bitcast_convert_type(msf, jnp.int32),
                jnp.float32)
            recip = seed
            for _ in range(3):
                recip = recip * (jnp.float32(2.0) - msf * recip)
            nz01 = jnp.where(step == 0, 0, 1)

            def lut_body(t, c):
                ce = hist[pl.ds(t * L, L)]
                x = ce + s2
                qi = (x.astype(jnp.float32) * recip).astype(jnp.int32)
                r = qi * ms
                qi = qi - jnp.where(r > x, 1, 0)
                qi = qi + jnp.where(r + ms <= x, 1, 0)
                lutv = jnp.clip(qi, 0, 255)
                vbase = t * L + iota16
                lutv = vbase + (lutv - vbase) * nz01
                lut[pl.ds(t * L, L)] = lutv.astype(jnp.float32)
                return c
            lax.fori_loop(0, NBINS // L, lut_body, 0)

            for c in range(nchunks):
                buf = ibufs[c % 2]
                obuf = obufs[c % 2]
                off = pl.multiple_of(base + c * CHUNK, CHUNK)
                pltpu.sync_copy(img_hbm.at[pl.ds(off, CHUNK)], buf)

                def gather_body(i, cc):
                    v = buf[pl.ds(i * L, L)]
                    idx = jnp.clip(v, 0.0, 255.0).astype(jnp.int32)
                    obuf[pl.ds(i * L, L)] = plsc.load_gather(lut, [idx])
                    return cc
                lax.fori_loop(0, vecs, gather_body, 0, unroll=4)
                pltpu.sync_copy(obuf, out_hbm.at[pl.ds(off, CHUNK)])

            return 0

        lax.fori_loop(0, planes_per_w, per_plane, 0)

    return eq_kernel


def kernel(img, target):
    B, C, H, W = img.shape
    n_planes = B * C
    n_pix = H * W
    flat = img.reshape(n_planes * n_pix)
    out = _make_equalize(n_planes, n_pix)(flat)
    return out.reshape(B, C, H, W), target


# speedup vs baseline: 883.1365x; 4.3619x over previous
import functools

import jax
import jax.numpy as jnp
from jax import lax
from jax.experimental import pallas as pl
from jax.experimental.pallas import tpu as pltpu
from jax.experimental.pallas import tpu_sc as plsc

L = 16
NBINS = 256
NWORKERS = 32
CHUNK = 16384


def _make_equalize(n_planes: int, n_pix: int):
    assert n_planes % NWORKERS == 0
    assert n_pix % CHUNK == 0
    planes_per_w = n_planes // NWORKERS
    nchunks = n_pix // CHUNK
    vecs = CHUNK // L

    mesh = plsc.VectorSubcoreMesh(core_axis_name="c", subcore_axis_name="s")

    @functools.partial(
        pl.kernel,
        out_type=jax.ShapeDtypeStruct((n_planes * n_pix,), jnp.float32),
        mesh=mesh,
        compiler_params=pltpu.CompilerParams(needs_layout_passes=False),
        scratch_types=[
            pltpu.VMEM((CHUNK,), jnp.float32),
            pltpu.VMEM((CHUNK,), jnp.float32),
            pltpu.VMEM((CHUNK,), jnp.float32),
            pltpu.VMEM((CHUNK,), jnp.float32),
            pltpu.VMEM((L * NBINS,), jnp.int32),
            pltpu.VMEM((NBINS,), jnp.float32),
            pltpu.SemaphoreType.DMA((2,)),
            pltpu.SemaphoreType.DMA((2,)),
        ],
    )
    def eq_kernel(img_hbm, out_hbm, in_a, in_b, out_a, out_b, hist, lut,
                  isems, osems):
        wid = lax.axis_index("s") * 2 + lax.axis_index("c")
        iota16 = lax.iota(jnp.int32, L)
        lane_base = iota16 * NBINS
        ones = jnp.ones((L,), jnp.int32)
        ibufs = [in_a, in_b]
        obufs = [out_a, out_b]

        def per_plane(j, _):
            plane = wid * planes_per_w + j
            base = plane * n_pix

            def in_cp(c):
                off = pl.multiple_of(base + c * CHUNK, CHUNK)
                return pltpu.make_async_copy(
                    img_hbm.at[pl.ds(off, CHUNK)], ibufs[c % 2],
                    isems.at[c % 2])

            def out_cp(c):
                off = pl.multiple_of(base + c * CHUNK, CHUNK)
                return pltpu.make_async_copy(
                    obufs[c % 2], out_hbm.at[pl.ds(off, CHUNK)],
                    osems.at[c % 2])

            in_cp(0).start()

            def zero_body(t, c):
                hist[pl.ds(t * L, L)] = jnp.zeros((L,), jnp.int32)
                return c
            lax.fori_loop(0, (L * NBINS) // L, zero_body, 0)

            for c in range(nchunks):
                if c + 1 < nchunks:
                    in_cp(c + 1).start()
                in_cp(c).wait()
                buf = ibufs[c % 2]

                @plsc.parallel_loop(0, CHUNK, step=L, unroll=8)
                def hist_body(i):
                    v = buf[pl.ds(i, L)]
                    idx = jnp.clip(v, 0.0, 255.0).astype(jnp.int32)
                    plsc.addupdate_scatter(hist, [idx + lane_base], ones)

            in_cp(0).start()
            in_cp(1).start()

            def merge_body(t, carry):
                csum, mvec = carry
                acc = hist[pl.ds(t * L, L)]
                for ln in range(1, L):
                    acc = acc + hist[pl.ds(ln * NBINS + t * L, L)]
                inc = plsc.cumsum(acc) + csum
                hist[pl.ds(t * L, L)] = inc - acc
                mvec = jnp.maximum(mvec, jnp.where(inc < n_pix, inc, 0))
                return (csum + jnp.sum(acc), mvec)

            _, mvec = lax.fori_loop(
                0, NBINS // L, merge_body,
                (jnp.int32(0), jnp.zeros((L,), jnp.int32)))
            m = jnp.max(mvec)

            q = (m.astype(jnp.float32) * jnp.float32(1.0 / 255.0)) \
                .astype(jnp.int32)
            q = q - jnp.where(q * 255 > m, 1, 0)
            q = q + jnp.where((q + 1) * 255 <= m, 1, 0)
            step = q
            s2 = step >> 1
            ms = jnp.maximum(step, 1)
            msf = ms.astype(jnp.float32)
            seed = lax.bitcast_convert_type(
                jnp.int32(0x7EF477D5)
                - lax.bitcast_convert_type(msf, jnp.int32),
                jnp.float32)
            recip = seed
            for _ in range(3):
                recip = recip * (jnp.float32(2.0) - msf * recip)
            nz01 = jnp.where(step == 0, 0, 1)

            def lut_body(t, c):
                ce = hist[pl.ds(t * L, L)]
                x = ce + s2
                qi = (x.astype(jnp.float32) * recip).astype(jnp.int32)
                r = qi * ms
                qi = qi - jnp.where(r > x, 1, 0)
                qi = qi + jnp.where(r + ms <= x, 1, 0)
                lutv = jnp.clip(qi, 0, 255)
                vbase = t * L + iota16
                lutv = vbase + (lutv - vbase) * nz01
                lut[pl.ds(t * L, L)] = lutv.astype(jnp.float32)
                return c
            lax.fori_loop(0, NBINS // L, lut_body, 0)

            for c in range(nchunks):
                buf = ibufs[c % 2]
                obuf = obufs[c % 2]
                in_cp(c).wait()
                if c >= 2:
                    out_cp(c - 2).wait()

                @plsc.parallel_loop(0, CHUNK, step=L, unroll=8)
                def gather_body(i):
                    v = buf[pl.ds(i, L)]
                    idx = jnp.clip(v, 0.0, 255.0).astype(jnp.int32)
                    obuf[pl.ds(i, L)] = plsc.load_gather(lut, [idx])

                out_cp(c).start()
                if c + 2 < nchunks:
                    in_cp(c + 2).start()

            out_cp(nchunks - 2).wait()
            out_cp(nchunks - 1).wait()
            return 0

        lax.fori_loop(0, planes_per_w, per_plane, 0)

    return eq_kernel


def kernel(img, target):
    B, C, H, W = img.shape
    n_planes = B * C
    n_pix = H * W
    flat = img.reshape(n_planes * n_pix)
    out = _make_equalize(n_planes, n_pix)(flat)
    return out.reshape(B, C, H, W), target


# speedup vs baseline: 1773.8427x; 2.0086x over previous
import functools

import jax
import jax.numpy as jnp
from jax import lax
from jax.experimental import pallas as pl
from jax.experimental.pallas import tpu as pltpu
from jax.experimental.pallas import tpu_sc as plsc

L = 16
NBINS = 256
NWORKERS = 32
CHUNK = 16384


def _make_equalize(n_planes: int, h: int, w: int):
    n_pix = h * w
    assert n_planes % NWORKERS == 0
    assert n_pix % CHUNK == 0 and CHUNK % w == 0 and w % L == 0
    rows = CHUNK // w
    planes_per_w = n_planes // NWORKERS
    nchunks = n_pix // CHUNK
    wshift = w.bit_length() - 1
    assert w == 1 << wshift

    mesh = plsc.VectorSubcoreMesh(core_axis_name="c", subcore_axis_name="s")

    @functools.partial(
        pl.kernel,
        out_type=jax.ShapeDtypeStruct((n_planes, h, w), jnp.float32),
        mesh=mesh,
        compiler_params=pltpu.CompilerParams(
            needs_layout_passes=False, use_tc_tiling_on_sc=True),
        scratch_types=[
            pltpu.VMEM((rows, w), jnp.float32),
            pltpu.VMEM((rows, w), jnp.float32),
            pltpu.VMEM((rows, w), jnp.float32),
            pltpu.VMEM((rows, w), jnp.float32),
            pltpu.VMEM((L * NBINS,), jnp.int32),
            pltpu.VMEM((NBINS,), jnp.float32),
            pltpu.SemaphoreType.DMA((2,)),
            pltpu.SemaphoreType.DMA((2,)),
        ],
    )
    def eq_kernel(img_hbm, out_hbm, in_a, in_b, out_a, out_b, hist, lut,
                  isems, osems):
        wid = lax.axis_index("s") * 2 + lax.axis_index("c")
        iota16 = lax.iota(jnp.int32, L)
        lane_base = iota16 * NBINS
        ones = jnp.ones((L,), jnp.int32)
        ibufs = [in_a, in_b]
        obufs = [out_a, out_b]

        def per_plane(j, _):
            plane = wid * planes_per_w + j

            def in_cp(c):
                return pltpu.make_async_copy(
                    img_hbm.at[plane, pl.ds(c * rows, rows), :],
                    ibufs[c % 2], isems.at[c % 2])

            def out_cp(c):
                return pltpu.make_async_copy(
                    obufs[c % 2],
                    out_hbm.at[plane, pl.ds(c * rows, rows), :],
                    osems.at[c % 2])

            in_cp(0).start()

            def zero_body(t, c):
                hist[pl.ds(t * L, L)] = jnp.zeros((L,), jnp.int32)
                return c
            lax.fori_loop(0, (L * NBINS) // L, zero_body, 0)

            for c in range(nchunks):
                if c + 1 < nchunks:
                    in_cp(c + 1).start()
                in_cp(c).wait()
                buf = ibufs[c % 2]

                @plsc.parallel_loop(0, CHUNK, step=L, unroll=8)
                def hist_body(i):
                    v = buf[i >> wshift, pl.ds(i & (w - 1), L)]
                    idx = jnp.clip(v, 0.0, 255.0).astype(jnp.int32)
                    plsc.addupdate_scatter(hist, [idx + lane_base], ones)

            in_cp(0).start()
            in_cp(1).start()

            def merge_body(t, carry):
                csum, mvec = carry
                acc = hist[pl.ds(t * L, L)]
                for ln in range(1, L):
                    acc = acc + hist[pl.ds(ln * NBINS + t * L, L)]
                inc = plsc.cumsum(acc) + csum
                hist[pl.ds(t * L, L)] = inc - acc
                mvec = jnp.maximum(mvec, jnp.where(inc < n_pix, inc, 0))
                return (csum + jnp.sum(acc), mvec)

            _, mvec = lax.fori_loop(
                0, NBINS // L, merge_body,
                (jnp.int32(0), jnp.zeros((L,), jnp.int32)))
            m = jnp.max(mvec)

            q = (m.astype(jnp.float32) * jnp.float32(1.0 / 255.0)) \
                .astype(jnp.int32)
            q = q - jnp.where(q * 255 > m, 1, 0)
            q = q + jnp.where((q + 1) * 255 <= m, 1, 0)
            step = q
            s2 = step >> 1
            ms = jnp.maximum(step, 1)
            msf = ms.astype(jnp.float32)
            seed = lax.bitcast_convert_type(
                jnp.int32(0x7EF477D5)
                - lax.bitcast_convert_type(msf, jnp.int32),
                jnp.float32)
            recip = seed
            for _ in range(3):
                recip = recip * (jnp.float32(2.0) - msf * recip)
            nz01 = jnp.where(step == 0, 0, 1)

            def lut_body(t, c):
                ce = hist[pl.ds(t * L, L)]
                x = ce + s2
                qi = (x.astype(jnp.float32) * recip).astype(jnp.int32)
                r = qi * ms
                qi = qi - jnp.where(r > x, 1, 0)
                qi = qi + jnp.where(r + ms <= x, 1, 0)
                lutv = jnp.clip(qi, 0, 255)
                vbase = t * L + iota16
                lutv = vbase + (lutv - vbase) * nz01
                lut[pl.ds(t * L, L)] = lutv.astype(jnp.float32)
                return c
            lax.fori_loop(0, NBINS // L, lut_body, 0)

            for c in range(nchunks):
                buf = ibufs[c % 2]
                obuf = obufs[c % 2]
                in_cp(c).wait()
                if c >= 2:
                    out_cp(c - 2).wait()

                @plsc.parallel_loop(0, CHUNK, step=L, unroll=8)
                def gather_body(i):
                    r = i >> wshift
                    cc = i & (w - 1)
                    v = buf[r, pl.ds(cc, L)]
                    idx = jnp.clip(v, 0.0, 255.0).astype(jnp.int32)
                    obuf[r, pl.ds(cc, L)] = plsc.load_gather(lut, [idx])

                out_cp(c).start()
                if c + 2 < nchunks:
                    in_cp(c + 2).start()

            out_cp(nchunks - 2).wait()
            out_cp(nchunks - 1).wait()
            return 0

        lax.fori_loop(0, planes_per_w, per_plane, 0)

    return eq_kernel


def kernel(img, target):
    B, C, H, W = img.shape
    n_planes = B * C
    flat = img.reshape(n_planes, H, W)
    out = _make_equalize(n_planes, H, W)(flat)
    return out.reshape(B, C, H, W), target
